# baseline (speedup 1.0000x reference)
import functools

import jax
import jax.numpy as jnp
from jax import lax
from jax.experimental import pallas as pl
from jax.experimental.pallas import tpu as pltpu
from jax.experimental.pallas import tpu_sc as plsc

_D = 300
_E = 64
_NC = 2
_NS = 16
_NW = _NC * _NS
_BN = 4096
_NBLK = 123
_OFF = (_NBLK - 1) * _BN
_BB = 256


def _tc_project_table(tt, w):
    def mm(x1_ref, x2_ref, w_ref, o_ref):
        ww = w_ref[...]
        a1 = lax.dot_general(ww, x1_ref[...], (((1,), (0,)), ((), ())),
                             preferred_element_type=jnp.float32)
        a2 = lax.dot_general(ww, x2_ref[...], (((1,), (0,)), ((), ())),
                             preferred_element_type=jnp.float32)
        o_ref[...] = jnp.concatenate([a1.T, a2.T], axis=1)

    return pl.pallas_call(
        mm,
        grid=(_NBLK,),
        in_specs=[
            pl.BlockSpec((_D, _BN), lambda i: (0, i)),
            pl.BlockSpec((_D, _BN), lambda i: (0, i + _NBLK - 1)),
            pl.BlockSpec((_E, _D), lambda i: (0, 0)),
        ],
        out_specs=pl.BlockSpec((_BN, 2 * _E), lambda i: (i, 0)),
        out_shape=jax.ShapeDtypeStruct((_NBLK * _BN, 2 * _E), jnp.float32),
    )(tt, tt, w)


def _sc_gather(packed, idx):
    rows = idx.shape[0]
    bpw = rows // _NW
    fchunk = 160
    nchunk = bpw // fchunk
    mesh = plsc.VectorSubcoreMesh(core_axis_name="c", subcore_axis_name="s")

    @functools.partial(
        pl.kernel,
        mesh=mesh,
        out_type=jax.ShapeDtypeStruct((rows * _E,), jnp.float32),
        scratch_types=[
            pltpu.VMEM((bpw,), jnp.int32),
            pltpu.VMEM((fchunk * 2 * _E,), jnp.float32),
            pltpu.VMEM((fchunk * 2 * _E,), jnp.float32),
            pltpu.VMEM((fchunk * _E,), jnp.float32),
            pltpu.VMEM((fchunk * _E,), jnp.float32),
            pltpu.SemaphoreType.DMA,
            pltpu.SemaphoreType.DMA,
            pltpu.SemaphoreType.DMA,
        ],
    )
    def gather_kernel(idx_hbm, tab_hbm, out_hbm, idx_v, buf0, buf1,
                      obuf0, obuf1, sem_g, sem_o0, sem_o1):
        wid = lax.axis_index("s") * _NC + lax.axis_index("c")
        fbase = pl.multiple_of(wid * bpw, fchunk)
        pltpu.sync_copy(idx_hbm.at[pl.ds(fbase, bpw)], idx_v)
        bufs = (buf0, buf1)
        obufs = (obuf0, obuf1)
        sem_os = (sem_o0, sem_o1)

        def outer(i, _):
            for b2 in range(2):
                buf = bufs[b2]
                obuf = obufs[b2]
                sem_ob = sem_os[b2]
                c = i * 2 + b2

                @pl.when(i > 0)
                def _drain():
                    pltpu.make_async_copy(
                        tab_hbm.at[pl.ds(0, fchunk * _E)], obuf,
                        sem_ob).wait()

                def body(g, _, c=c, buf=buf):
                    off = pl.multiple_of(c * fchunk + g * 16, 16)
                    vec = idx_v[pl.ds(off, 16)]
                    vec = (vec - jnp.where(vec >= _OFF, _OFF, 0)) * (2 * _E)
                    for e in range(16):
                        pltpu.async_copy(
                            tab_hbm.at[pl.ds(
                                pl.multiple_of(vec[e], 2 * _E), 2 * _E)],
                            buf.at[pl.ds(
                                pl.multiple_of((g * 16 + e) * 2 * _E,
                                               2 * _E), 2 * _E)], sem_g)
                    return 0
                lax.fori_loop(0, fchunk // 16, body, 0)
                pltpu.make_async_copy(
                    tab_hbm.at[pl.ds(0, fchunk * 2 * _E)], buf, sem_g).wait()

                def pick(g, _, c=c, buf=buf, obuf=obuf):
                    off = pl.multiple_of(c * fchunk + g * 16, 16)
                    pv = jnp.where(idx_v[pl.ds(off, 16)] >= _OFF, 1.0, 0.0)
                    for e in range(16):
                        j = g * 16 + e
                        for k in range(_E // 16):
                            lo = buf[pl.ds(
                                pl.multiple_of(j * 2 * _E + k * 16, 16), 16)]
                            hi = buf[pl.ds(
                                pl.multiple_of(j * 2 * _E + _E + k * 16, 16),
                                16)]
                            obuf[pl.ds(
                                pl.multiple_of(j * _E + k * 16, 16), 16)] = (
                                    lo + (hi - lo) * pv[e])
                    return 0
                lax.fori_loop(0, fchunk // 16, pick, 0)
                pltpu.async_copy(
                    obuf, out_hbm.at[pl.ds(
                        pl.multiple_of((fbase + c * fchunk) * _E,
                                       fchunk * _E), fchunk * _E)],
                    sem_ob)
            return 0

        lax.fori_loop(0, nchunk // 2, outer, 0)
        for b2 in range(2):
            pltpu.make_async_copy(
                tab_hbm.at[pl.ds(0, fchunk * _E)], obufs[b2],
                sem_os[b2]).wait()

    return gather_kernel(idx, packed.reshape(-1))


def kernel(inputs, table, W):
    b, l = inputs.shape
    idx = inputs.reshape(-1)
    packed = _tc_project_table(table.T, W)
    out = _sc_gather(packed, idx)
    return out.reshape(b, l, _E)
